# Initial kernel scaffold; baseline (speedup 1.0000x reference)
import jax
import jax.numpy as jnp
from jax.experimental import pallas as pl


def kernel(x, W, A, B):
    raise NotImplementedError("write your pallas kernel here")


# speedup vs baseline: 4.7045x; 4.7045x over previous
import functools

import jax
import jax.numpy as jnp
from jax import lax
from jax.experimental import pallas as pl
from jax.experimental.pallas import tpu as pltpu
from jax.experimental.pallas import tpu_sc as plsc

NUM_ROWS = 100000
DIM = 64
RANK = 8

NC = 2
NS = 16
NW = NC * NS
IDX_TOTAL = 204800
PER_W = IDX_TOTAL // NW
BLK_I = 128
STREAMS_PER_GROUP = 5
GROUP = STREAMS_PER_GROUP * BLK_I
NGROUPS = PER_W // GROUP

FUSE_BLK = 2000


def _fuse_body(w_ref, b_ref, a_ref, t_ref):
    t_ref[...] = w_ref[...] + jnp.dot(
        b_ref[...], a_ref[...], preferred_element_type=jnp.float32)


def _fuse_table(W, A, B):
    grid = (NUM_ROWS // FUSE_BLK,)
    return pl.pallas_call(
        _fuse_body,
        grid=grid,
        in_specs=[
            pl.BlockSpec((FUSE_BLK, DIM), lambda i: (i, 0)),
            pl.BlockSpec((FUSE_BLK, RANK), lambda i: (i, 0)),
            pl.BlockSpec((RANK, DIM), lambda i: (0, 0)),
        ],
        out_specs=pl.BlockSpec((FUSE_BLK, DIM), lambda i: (i, 0)),
        out_shape=jax.ShapeDtypeStruct((NUM_ROWS, DIM), jnp.float32),
    )(W, B, A)


def _sc_gather(table, idx3):
    mesh = plsc.VectorSubcoreMesh(core_axis_name="c", subcore_axis_name="s")

    @functools.partial(
        pl.kernel,
        mesh=mesh,
        compiler_params=pltpu.CompilerParams(use_tc_tiling_on_sc=False),
        out_type=jax.ShapeDtypeStruct(
            (NW, NGROUPS, STREAMS_PER_GROUP, BLK_I, DIM), jnp.float32),
        scratch_types=[
            pltpu.VMEM((NGROUPS * STREAMS_PER_GROUP, BLK_I), jnp.int32),
            pltpu.VMEM((STREAMS_PER_GROUP, BLK_I, DIM), jnp.float32),
            pltpu.SemaphoreType.DMA,
        ],
    )
    def k(table_hbm, idx_hbm, out_hbm, idx_v, rows_v, gsem):
        wid = lax.axis_index("s") * NC + lax.axis_index("c")
        pltpu.sync_copy(idx_hbm.at[wid], idx_v)

        def body(g, carry):
            descs = []
            for s in range(STREAMS_PER_GROUP):
                descs.append(pltpu.async_copy(
                    table_hbm.at[idx_v.at[g * STREAMS_PER_GROUP + s]],
                    rows_v.at[s], gsem))
            for d in descs:
                d.wait()
            pltpu.sync_copy(rows_v, out_hbm.at[wid, g])
            return carry

        lax.fori_loop(0, NGROUPS, body, 0)

    return k(table, idx3)


def kernel(x, W, A, B):
    T = _fuse_table(W, A, B)
    idx3 = x.reshape(NW, NGROUPS * STREAMS_PER_GROUP, BLK_I)
    out = _sc_gather(T, idx3)
    return out.reshape(4096, 50, DIM)
